# Initial kernel scaffold; baseline (speedup 1.0000x reference)
import functools

import jax
import jax.numpy as jnp
from jax import lax
from jax.experimental import pallas as pl
from jax.experimental.pallas import tpu as pltpu
from jax.experimental.pallas import tpu_sc as plsc

_NC = 2
_NS = 16
_NW = _NC * _NS
_LANES = 16
_IDX_W = 128


def _make_pool(V, D, B, LIN):
    items_per_w = B // _NW
    rows_per_w = items_per_w * LIN
    r_per_w = rows_per_w // _IDX_W
    ch_items = 128
    ch_rows = ch_items * LIN
    g_per_ch = ch_rows // _IDX_W
    n_chunks = items_per_w // ch_items

    mesh = plsc.VectorSubcoreMesh(core_axis_name="c", subcore_axis_name="s")

    @functools.partial(
        pl.kernel,
        out_type=jax.ShapeDtypeStruct((B, D), jnp.float32),
        mesh=mesh,
        scratch_types=[
            pltpu.VMEM((r_per_w, _IDX_W), jnp.int32),
            pltpu.VMEM((ch_rows, D), jnp.float32),
            pltpu.VMEM((ch_rows, D), jnp.float32),
            pltpu.VMEM((ch_items, D), jnp.float32),
            pltpu.VMEM((ch_items, D), jnp.float32),
            pltpu.SemaphoreType.DMA,
            pltpu.SemaphoreType.DMA,
            pltpu.SemaphoreType.DMA,
            pltpu.SemaphoreType.DMA,
        ],
    )
    def pool(table_h, idx_h, out_h, idx_v, rows0, rows1, outv0, outv1,
             sem_g0, sem_g1, sem_o0, sem_o1):
        wid = lax.axis_index("s") * _NC + lax.axis_index("c")
        pltpu.sync_copy(idx_h.at[wid], idx_v)

        rows_bufs = (rows0, rows1)
        out_bufs = (outv0, outv1)
        gather_sems = (sem_g0, sem_g1)
        out_sems = (sem_o0, sem_o1)

        def start_gathers(g):
            p = g % 2
            buf = rows_bufs[p]
            handles = []
            for j in range(g_per_ch):
                h = pltpu.async_copy(
                    table_h.at[idx_v.at[g * g_per_ch + j]],
                    buf.at[pl.ds(j * _IDX_W, _IDX_W)],
                    gather_sems[p],
                )
                handles.append(h)
            return handles

        def compute(g):
            rows = rows_bufs[g % 2]
            outv = out_bufs[g % 2]

            @pl.loop(0, ch_items, unroll=2)
            def _(i):
                base = i * LIN
                for s in range(D // _LANES):
                    sl = pl.ds(s * _LANES, _LANES)
                    acc = rows[base, sl]
                    for j in range(1, LIN):
                        acc = acc + rows[base + j, sl]
                    outv[i, sl] = acc

        gh = start_gathers(0)
        oh = [None, None]
        for g in range(n_chunks):
            nxt = start_gathers(g + 1) if g + 1 < n_chunks else None
            for h in gh:
                h.wait()
            if oh[g % 2] is not None:
                oh[g % 2].wait()
            compute(g)
            oh[g % 2] = pltpu.async_copy(
                out_bufs[g % 2],
                out_h.at[pl.ds(wid * items_per_w + g * ch_items, ch_items)],
                out_sems[g % 2],
            )
            gh = nxt
        for h in oh:
            if h is not None:
                h.wait()

    return pool


def _mlp_body(x_ref, w1_ref, b1_ref, w2_ref, b2_ref, w3_ref, b3_ref, o_ref):
    x = x_ref[...]
    h = jnp.dot(x, w1_ref[...], preferred_element_type=jnp.float32) + b1_ref[...]
    h = jnp.maximum(h, 0.0)
    h = jnp.dot(h, w2_ref[...], preferred_element_type=jnp.float32) + b2_ref[...]
    h = jnp.maximum(h, 0.0)
    o_ref[...] = jnp.sum(h * w3_ref[...], axis=1) + b3_ref[0]


def _make_mlp(B, D, H):
    blk = 1024
    grid = (B // blk,)
    return pl.pallas_call(
        _mlp_body,
        grid=grid,
        in_specs=[
            pl.BlockSpec((blk, D), lambda i: (i, 0)),
            pl.BlockSpec((D, H), lambda i: (0, 0)),
            pl.BlockSpec((1, H), lambda i: (0, 0)),
            pl.BlockSpec((H, H), lambda i: (0, 0)),
            pl.BlockSpec((1, H), lambda i: (0, 0)),
            pl.BlockSpec((1, H), lambda i: (0, 0)),
            pl.BlockSpec(memory_space=pltpu.SMEM),
        ],
        out_specs=pl.BlockSpec((blk,), lambda i: (i,)),
        out_shape=jax.ShapeDtypeStruct((B,), jnp.float32),
    )


def kernel(player_indices, table, W1, b1, W2, b2, W3, b3):
    B, LIN = player_indices.shape
    V, D = table.shape
    H = W1.shape[1]

    idx = player_indices.astype(jnp.int32).reshape(
        _NW, (B * LIN) // (_NW * _IDX_W), _IDX_W)
    pooled = _make_pool(V, D, B, LIN)(table, idx)

    W1s = W1 * (1.0 / LIN)
    out = _make_mlp(B, D, H)(
        pooled, W1s, b1.reshape(1, H), W2, b2.reshape(1, H),
        W3.reshape(1, H), b3)
    return out


# speedup vs baseline: 1.7026x; 1.7026x over previous
import functools

import jax
import jax.numpy as jnp
from jax import lax
from jax.experimental import pallas as pl
from jax.experimental.pallas import tpu as pltpu
from jax.experimental.pallas import tpu_sc as plsc

_NC = 2
_NS = 16
_NW = _NC * _NS
_LANES = 16
_IDX_W = 128


def _make_pool(V, D, B, LIN):
    items_per_w = B // _NW
    rows_per_w = items_per_w * LIN
    r_per_w = rows_per_w // _IDX_W
    ch_items = 128
    ch_rows = ch_items * LIN
    g_per_ch = ch_rows // _IDX_W
    n_chunks = items_per_w // ch_items

    mesh = plsc.VectorSubcoreMesh(core_axis_name="c", subcore_axis_name="s")

    @functools.partial(
        pl.kernel,
        out_type=jax.ShapeDtypeStruct((B, D), jnp.float32),
        mesh=mesh,
        compiler_params=pltpu.CompilerParams(use_tc_tiling_on_sc=False),
        scratch_types=[
            pltpu.VMEM((r_per_w, _IDX_W), jnp.int32),
            pltpu.VMEM((ch_rows, D), jnp.float32),
            pltpu.VMEM((ch_rows, D), jnp.float32),
            pltpu.VMEM((ch_items, D), jnp.float32),
            pltpu.VMEM((ch_items, D), jnp.float32),
            pltpu.SemaphoreType.DMA,
            pltpu.SemaphoreType.DMA,
            pltpu.SemaphoreType.DMA,
            pltpu.SemaphoreType.DMA,
        ],
    )
    def pool(table_h, idx_h, out_h, idx_v, rows0, rows1, outv0, outv1,
             sem_g0, sem_g1, sem_o0, sem_o1):
        wid = lax.axis_index("s") * _NC + lax.axis_index("c")
        pltpu.sync_copy(idx_h.at[wid], idx_v)

        rows_bufs = (rows0, rows1)
        out_bufs = (outv0, outv1)
        gather_sems = (sem_g0, sem_g1)
        out_sems = (sem_o0, sem_o1)

        def start_gathers(g):
            p = g % 2
            buf = rows_bufs[p]
            handles = []
            for j in range(g_per_ch):
                h = pltpu.async_copy(
                    table_h.at[idx_v.at[g * g_per_ch + j]],
                    buf.at[pl.ds(j * _IDX_W, _IDX_W)],
                    gather_sems[p],
                )
                handles.append(h)
            return handles

        def compute(g):
            rows = rows_bufs[g % 2]
            outv = out_bufs[g % 2]

            @pl.loop(0, ch_items, unroll=2)
            def _(i):
                base = i * LIN
                for s in range(D // _LANES):
                    sl = pl.ds(s * _LANES, _LANES)
                    acc = rows[base, sl]
                    for j in range(1, LIN):
                        acc = acc + rows[base + j, sl]
                    outv[i, sl] = acc

        gh = start_gathers(0)
        oh = [None, None]
        for g in range(n_chunks):
            nxt = start_gathers(g + 1) if g + 1 < n_chunks else None
            for h in gh:
                h.wait()
            if oh[g % 2] is not None:
                oh[g % 2].wait()
            compute(g)
            oh[g % 2] = pltpu.async_copy(
                out_bufs[g % 2],
                out_h.at[pl.ds(wid * items_per_w + g * ch_items, ch_items)],
                out_sems[g % 2],
            )
            gh = nxt
        for h in oh:
            if h is not None:
                h.wait()

    return pool


def _mlp_body(x_ref, w1_ref, b1_ref, w2_ref, b2_ref, w3_ref, b3_ref, o_ref):
    x = x_ref[...]
    h = jnp.dot(x, w1_ref[...], preferred_element_type=jnp.float32) + b1_ref[...]
    h = jnp.maximum(h, 0.0)
    h = jnp.dot(h, w2_ref[...], preferred_element_type=jnp.float32) + b2_ref[...]
    h = jnp.maximum(h, 0.0)
    o_ref[...] = jnp.sum(h * w3_ref[...], axis=1) + b3_ref[0]


def _make_mlp(B, D, H):
    blk = 1024
    grid = (B // blk,)
    return pl.pallas_call(
        _mlp_body,
        grid=grid,
        in_specs=[
            pl.BlockSpec((blk, D), lambda i: (i, 0)),
            pl.BlockSpec((D, H), lambda i: (0, 0)),
            pl.BlockSpec((1, H), lambda i: (0, 0)),
            pl.BlockSpec((H, H), lambda i: (0, 0)),
            pl.BlockSpec((1, H), lambda i: (0, 0)),
            pl.BlockSpec((1, H), lambda i: (0, 0)),
            pl.BlockSpec(memory_space=pltpu.SMEM),
        ],
        out_specs=pl.BlockSpec((blk,), lambda i: (i,)),
        out_shape=jax.ShapeDtypeStruct((B,), jnp.float32),
    )


def kernel(player_indices, table, W1, b1, W2, b2, W3, b3):
    B, LIN = player_indices.shape
    V, D = table.shape
    H = W1.shape[1]

    idx = player_indices.astype(jnp.int32).reshape(
        _NW, (B * LIN) // (_NW * _IDX_W), _IDX_W)
    pooled = _make_pool(V, D, B, LIN)(table, idx)

    W1s = W1 * (1.0 / LIN)
    out = _make_mlp(B, D, H)(
        pooled, W1s, b1.reshape(1, H), W2, b2.reshape(1, H),
        W3.reshape(1, H), b3)
    return out
